# Initial kernel scaffold; baseline (speedup 1.0000x reference)
import jax
import jax.numpy as jnp
from jax.experimental import pallas as pl


def kernel(x, weight, bias):
    raise NotImplementedError("write your pallas kernel here")


# speedup vs baseline: 1.5914x; 1.5914x over previous
import functools

import jax
import jax.numpy as jnp
from jax.experimental import pallas as pl
from jax.experimental.pallas import tpu as pltpu


_DROWS = 8


def _conv_kernel(x_ref, w_ref, b_ref, o_ref, *, ho, wo):
    c, drows, hw = x_ref.shape[1], x_ref.shape[2], x_ref.shape[3]
    howo = o_ref.shape[2]

    x2 = x_ref[0].reshape(c * drows, hw)
    o = jax.lax.dot_general(
        w_ref[...], x2,
        dimension_numbers=(((0,), (0,)), ((), ())),
        preferred_element_type=jnp.float32,
    )

    o4 = o.reshape(4, 4, 8, hw)

    def tap(kh, kw):
        a = o4[:, 2 * kh + kw]
        a = a.reshape(4, 8, ho, 2, wo, 2)
        return a[:, :, :, kh, :, kw]

    s = tap(0, 0) + tap(0, 1) + tap(1, 0) + tap(1, 1)
    res = jnp.transpose(s, (0, 2, 3, 1)) + b_ref[0]
    o_ref[...] = res.reshape(1, 4, howo, 8)


def kernel(x, weight, bias):
    n, c, d, h, w = x.shape
    cout = weight.shape[0]
    do, ho, wo = d // 2, h // 2, w // 2
    hw = h * w
    dblk = d // _DROWS

    xv = x.reshape(n, c, d, hw)

    w5 = jnp.transpose(weight, (1, 2, 3, 4, 0))
    w5 = w5.reshape(c, 2, 4 * cout)
    sel = jnp.zeros((_DROWS, 4, 2), dtype=x.dtype)
    dd = jnp.arange(_DROWS)
    sel = sel.at[dd, dd // 2, dd % 2].set(1.0)
    w_big = jnp.einsum('ckm,djk->cdjm', w5, sel)
    w_big = w_big.reshape(c * _DROWS, 4 * 4 * cout)

    b2 = bias.astype(jnp.float32).reshape(1, cout)

    cost = pl.CostEstimate(
        flops=2 * (c * _DROWS) * (4 * 4 * cout) * hw * (n * dblk),
        transcendentals=0,
        bytes_accessed=x.size * 4 + w_big.size * 4 + n * do * ho * wo * cout * 4,
    )

    out = pl.pallas_call(
        functools.partial(_conv_kernel, ho=ho, wo=wo),
        out_shape=jax.ShapeDtypeStruct((n, do, ho * wo, cout), jnp.float32),
        grid=(n, dblk),
        in_specs=[
            pl.BlockSpec((1, c, _DROWS, hw), lambda i, j: (i, 0, j, 0)),
            pl.BlockSpec((c * _DROWS, 4 * 4 * cout), lambda i, j: (0, 0)),
            pl.BlockSpec((1, cout), lambda i, j: (0, 0)),
        ],
        out_specs=pl.BlockSpec((1, 4, ho * wo, cout), lambda i, j: (i, j, 0, 0)),
        compiler_params=pltpu.CompilerParams(
            dimension_semantics=("parallel", "parallel")),
        cost_estimate=cost,
    )(xv, w_big, b2)

    return out.reshape(n, do, ho, wo, cout)


# speedup vs baseline: 3.6205x; 2.2750x over previous
import functools

import numpy as np

import jax
import jax.numpy as jnp
from jax.experimental import pallas as pl
from jax.experimental.pallas import tpu as pltpu


_DROWS = 8


def _conv_kernel(x_ref, w_ref, sel_ref, b_ref, o_ref, *, cout):
    c, drows, hw = x_ref.shape[1], x_ref.shape[2], x_ref.shape[3]
    howo = o_ref.shape[2]

    x2 = x_ref[0].reshape(c * drows, hw).astype(jnp.bfloat16)
    o = jax.lax.dot_general(
        w_ref[...], x2,
        dimension_numbers=(((0,), (0,)), ((), ())),
        preferred_element_type=jnp.float32,
    )
    ob = o.astype(jnp.bfloat16)

    rows = 4 * cout
    s = None
    for t in range(4):
        part = jax.lax.dot_general(
            ob[rows * t:rows * (t + 1)],
            sel_ref[hw * t:hw * (t + 1)],
            dimension_numbers=(((1,), (0,)), ((), ())),
            preferred_element_type=jnp.float32,
        )
        s = part if s is None else s + part

    s3 = s.reshape(4, cout, howo)
    res = jnp.transpose(s3, (0, 2, 1)) + b_ref[0]
    o_ref[...] = res.reshape(1, 4, howo, cout)


def kernel(x, weight, bias):
    n, c, d, h, w = x.shape
    cout = weight.shape[0]
    do, ho, wo = d // 2, h // 2, w // 2
    hw = h * w
    howo = ho * wo
    dblk = d // _DROWS

    xv = x.reshape(n, c, d, hw)

    w5 = jnp.transpose(weight, (1, 2, 3, 4, 0))
    sel_d = np.zeros((_DROWS, 4, 2), dtype=np.float32)
    dd = np.arange(_DROWS)
    sel_d[dd, dd // 2, dd % 2] = 1.0
    w_big = jnp.einsum('ckxyo,djk->cdxyjo', w5, jnp.asarray(sel_d))
    w_big = w_big.reshape(c * _DROWS, 4 * 4 * cout)
    w_big = w_big.astype(jnp.bfloat16)

    sel_np = np.zeros((4, hw, howo), dtype=np.float32)
    ho_i, wo_i = np.meshgrid(np.arange(ho), np.arange(wo), indexing='ij')
    for kh in (0, 1):
        for kw in (0, 1):
            src = (2 * ho_i + kh) * w + 2 * wo_i + kw
            dst = ho_i * wo + wo_i
            sel_np[2 * kh + kw, src.ravel(), dst.ravel()] = 1.0
    sel = jnp.asarray(sel_np.reshape(4 * hw, howo), dtype=jnp.bfloat16)

    b2 = bias.astype(jnp.float32).reshape(1, cout)

    cost = pl.CostEstimate(
        flops=2 * (c * _DROWS) * (16 * cout) * hw * (n * dblk)
        + 2 * 4 * (4 * cout) * hw * howo * (n * dblk),
        transcendentals=0,
        bytes_accessed=x.size * 4 + w_big.size * 4 + sel.size * 4
        + n * do * howo * cout * 4,
    )

    out = pl.pallas_call(
        functools.partial(_conv_kernel, cout=cout),
        out_shape=jax.ShapeDtypeStruct((n, do, howo, cout), jnp.float32),
        grid=(n, dblk),
        in_specs=[
            pl.BlockSpec((1, c, _DROWS, hw), lambda i, j: (i, 0, j, 0)),
            pl.BlockSpec((c * _DROWS, 16 * cout), lambda i, j: (0, 0)),
            pl.BlockSpec((4 * hw, howo), lambda i, j: (0, 0)),
            pl.BlockSpec((1, cout), lambda i, j: (0, 0)),
        ],
        out_specs=pl.BlockSpec((1, 4, howo, cout), lambda i, j: (i, j, 0, 0)),
        compiler_params=pltpu.CompilerParams(
            dimension_semantics=("parallel", "parallel")),
        cost_estimate=cost,
    )(xv, w_big, sel, b2)

    return out.reshape(n, do, ho, wo, cout)
